# baseline (speedup 1.0000x reference)
import functools

import jax
import jax.numpy as jnp
from jax import lax
from jax.experimental import pallas as pl
from jax.experimental.pallas import tpu as pltpu
from jax.experimental.pallas import tpu_sc as plsc

_REP = 4
_ROWS = 2048
_D = 2048
_NC = 2
_NS = 16
_NW = _NC * _NS
_RPW = _ROWS // _NW
_C = 16

_mesh = plsc.VectorSubcoreMesh(core_axis_name="c", subcore_axis_name="s")


@functools.partial(
    pl.kernel,
    mesh=_mesh,
    out_type=jax.ShapeDtypeStruct((_ROWS, _REP, _D), jnp.float32),
    scratch_types=[
        pltpu.VMEM((_C, 1, _D), jnp.float32),
        pltpu.SemaphoreType.DMA,
    ],
)
def _upsample(x_hbm, out_hbm, buf, sem):
    wid = lax.axis_index("s") * _NC + lax.axis_index("c")
    base = wid * _RPW
    for i in range(_RPW // _C):
        b = base + i * _C
        pltpu.async_copy(x_hbm.at[pl.ds(b, _C)], buf, sem).wait()
        for r in range(_REP):
            pltpu.sync_copy(buf, out_hbm.at[pl.ds(b, _C), pl.ds(r, 1)])


def kernel(x, x_short):
    out3d = _upsample(x_short.reshape(_ROWS, 1, _D))
    return out3d.reshape(_ROWS * _REP, _D)


# speedup vs baseline: 1.0138x; 1.0138x over previous
import functools

import jax
import jax.numpy as jnp
from jax import lax
from jax.experimental import pallas as pl
from jax.experimental.pallas import tpu as pltpu
from jax.experimental.pallas import tpu_sc as plsc

_REP = 4
_ROWS = 2048
_D = 2048
_NC = 2
_NS = 16
_NW = _NC * _NS
_RPW = _ROWS // _NW
_C = 16

_mesh = plsc.VectorSubcoreMesh(core_axis_name="c", subcore_axis_name="s")


@functools.partial(
    pl.kernel,
    mesh=_mesh,
    out_type=jax.ShapeDtypeStruct((_ROWS, _REP, _D), jnp.float32),
    scratch_types=[
        pltpu.VMEM((_C, 1, _D), jnp.float32),
        pltpu.VMEM((_C, 1, _D), jnp.float32),
        pltpu.SemaphoreType.DMA,
        pltpu.SemaphoreType.DMA,
        pltpu.SemaphoreType.DMA,
        pltpu.SemaphoreType.DMA,
    ],
)
def _upsample(x_hbm, out_hbm, buf0, buf1, rsem0, rsem1, wsem0, wsem1):
    wid = lax.axis_index("s") * _NC + lax.axis_index("c")
    base = wid * _RPW
    bufs = (buf0, buf1)
    rsems = (rsem0, rsem1)
    wsems = (wsem0, wsem1)
    nch = _RPW // _C

    def rd(i):
        return pltpu.async_copy(
            x_hbm.at[pl.ds(base + i * _C, _C)], bufs[i % 2], rsems[i % 2])

    reads = {0: rd(0), 1: rd(1)}
    writes = {}
    for i in range(nch):
        bi = i % 2
        reads[i].wait()
        writes[i] = [
            pltpu.async_copy(
                bufs[bi], out_hbm.at[pl.ds(base + i * _C, _C), pl.ds(r, 1)],
                wsems[bi])
            for r in range(_REP)
        ]
        if i + 2 < nch:
            for w in writes[i]:
                w.wait()
            reads[i + 2] = rd(i + 2)
    for w in writes[nch - 2] + writes[nch - 1]:
        w.wait()


def kernel(x, x_short):
    out3d = _upsample(x_short.reshape(_ROWS, 1, _D))
    return out3d.reshape(_ROWS * _REP, _D)


# speedup vs baseline: 2.7736x; 2.7359x over previous
import functools

import jax
import jax.numpy as jnp
from jax import lax
from jax.experimental import pallas as pl
from jax.experimental.pallas import tpu as pltpu
from jax.experimental.pallas import tpu_sc as plsc

_REP = 4
_ROWS = 2048
_D = 2048
_NC = 2
_NS = 16
_NW = _NC * _NS
_RPW = _ROWS // _NW
_C = 16
_NCH = _RPW // _C

_mesh = plsc.VectorSubcoreMesh(core_axis_name="c", subcore_axis_name="s")


@functools.partial(
    pl.kernel,
    mesh=_mesh,
    out_type=jax.ShapeDtypeStruct((_ROWS * _REP, _D), jnp.float32),
    scratch_types=[
        pltpu.VMEM((_C, _D), jnp.float32),
        pltpu.VMEM((_C, _D), jnp.float32),
        pltpu.VMEM((_NCH * _REP, _C), jnp.int32),
        pltpu.SemaphoreType.DMA,
        pltpu.SemaphoreType.DMA,
        pltpu.SemaphoreType.DMA,
        pltpu.SemaphoreType.DMA,
    ],
)
def _upsample(x_hbm, out_hbm, buf0, buf1, idx, rsem0, rsem1, wsem0, wsem1):
    wid = lax.axis_index("s") * _NC + lax.axis_index("c")
    base = wid * _RPW
    bufs = (buf0, buf1)
    rsems = (rsem0, rsem1)
    wsems = (wsem0, wsem1)

    lanes = _REP * lax.iota(jnp.int32, _C)
    for i in range(_NCH):
        for r in range(_REP):
            idx[i * _REP + r] = _REP * base + _REP * _C * i + r + lanes

    def rd(i):
        return pltpu.async_copy(
            x_hbm.at[pl.ds(base + i * _C, _C)], bufs[i % 2], rsems[i % 2])

    reads = {0: rd(0), 1: rd(1)}
    writes = {}
    for i in range(_NCH):
        bi = i % 2
        reads[i].wait()
        writes[i] = [
            pltpu.async_copy(
                bufs[bi], out_hbm.at[idx.at[i * _REP + r]], wsems[bi])
            for r in range(_REP)
        ]
        if i + 2 < _NCH:
            for w in writes[i]:
                w.wait()
            reads[i + 2] = rd(i + 2)
    for w in writes[_NCH - 2] + writes[_NCH - 1]:
        w.wait()


def kernel(x, x_short):
    return _upsample(x_short)
